# baseline (speedup 1.0000x reference)
import functools

import jax
import jax.numpy as jnp
import numpy as np
from jax.experimental import pallas as pl
from jax.experimental.pallas import tpu as pltpu

_ROT = ((13, 15, 26, 6), (17, 29, 16, 24))
_KEY1 = 0
_KEY2 = 42
_TINY = np.float32(np.finfo(np.float32).tiny)


def _threefry_bits(x1):
    ks0 = jnp.uint32(_KEY1)
    ks1 = jnp.uint32(_KEY2)
    ks2 = jnp.uint32(_KEY1 ^ _KEY2 ^ 0x1BD11BDA)
    ks = (ks0, ks1, ks2)
    x0 = jnp.zeros_like(x1) + ks0
    for i in range(5):
        for r in _ROT[i % 2]:
            x0 = x0 + x1
            x1 = (x1 << r) | (x1 >> (32 - r))
            x1 = x1 ^ x0
        x0 = x0 + ks[(i + 1) % 3]
        x1 = x1 + ks[(i + 2) % 3] + jnp.uint32(i + 1)
    return x0 ^ x1


def _gumbel(bits):
    fb = (bits >> 9) | jnp.uint32(0x3F800000)
    u = jax.lax.bitcast_convert_type(fb, jnp.float32) - jnp.float32(1.0)
    return -jnp.log(-jnp.log(u))


def _kern(ns_ref, w_ref, p_ref, o_ref, *, rows, d_sub, d_lane, nd_mod, chunk, tail):
    i = pl.program_id(0)
    d = d_sub * d_lane
    d_io = (jax.lax.broadcasted_iota(jnp.uint32, (d_sub, d_lane), 0)
            * jnp.uint32(d_lane)
            + jax.lax.broadcasted_iota(jnp.uint32, (d_sub, d_lane), 1))
    for r in range(rows):
        n = i * rows + r
        ns = ns_ref[n]
        w = w_ref[r]
        logp = jnp.log(p_ref[r] + jnp.float32(1e-30))
        row_ct = d_io + jnp.uint32(d) * jnp.uint32(n) + jnp.uint32(_KEY2)

        def sample(s, rmax, *, masked):
            x1 = row_ct + s.astype(jnp.uint32) * jnp.uint32(nd_mod)
            v = _gumbel(_threefry_bits(x1)) + logp
            m = jnp.max(v, axis=(0, 1), keepdims=True)
            vm = v - m
            if masked:
                vm = jnp.where(s < ns, vm, jnp.float32(-jnp.inf))
            return jnp.maximum(rmax, vm)

        def main_body(j, rmax):
            for c in range(chunk):
                rmax = sample(j * chunk + c, rmax, masked=False)
            return rmax

        n_main = ns // chunk

        def tail_body(j, rmax):
            for c in range(tail):
                rmax = sample(n_main * chunk + j * tail + c, rmax,
                              masked=True)
            return rmax

        rmax = jax.lax.fori_loop(
            0, n_main, main_body,
            jnp.full((d_sub, d_lane), -jnp.inf, jnp.float32))
        rmax = jax.lax.fori_loop(
            0, pl.cdiv(ns - n_main * chunk, tail), tail_body, rmax)
        keep_f = (rmax == 0.0).astype(jnp.float32)
        nnz_c = jnp.sum(keep_f)
        nnz_w = jnp.sum((w != 0).astype(jnp.float32))
        gamma = jnp.where(
            ns >= 1,
            jnp.where(nnz_c < nnz_w, keep_f, jnp.ones_like(keep_f)),
            jnp.zeros_like(keep_f))
        o_ref[r] = gamma * w


def _sparsify(weight, probs, num_samples, rows=16, chunk=32, tail=8, interpret=False):
    n_rows, d = weight.shape
    d_sub = 8
    d_lane = d // d_sub
    w3 = weight.reshape(n_rows, d_sub, d_lane)
    p3 = probs.reshape(n_rows, d_sub, d_lane)
    ns = num_samples.astype(jnp.int32)
    out3 = pl.pallas_call(
        functools.partial(_kern, rows=rows, d_sub=d_sub, d_lane=d_lane,
                          nd_mod=(n_rows * d) % (2 ** 32), chunk=chunk, tail=tail),
        grid_spec=pltpu.PrefetchScalarGridSpec(
            num_scalar_prefetch=1,
            grid=(n_rows // rows,),
            in_specs=[
                pl.BlockSpec((rows, d_sub, d_lane), lambda i, s: (i, 0, 0)),
                pl.BlockSpec((rows, d_sub, d_lane), lambda i, s: (i, 0, 0)),
            ],
            out_specs=pl.BlockSpec((rows, d_sub, d_lane),
                                   lambda i, s: (i, 0, 0)),
        ),
        out_shape=jax.ShapeDtypeStruct((n_rows, d_sub, d_lane), jnp.float32),
        compiler_params=pltpu.CompilerParams(
            dimension_semantics=("parallel",)),
        interpret=interpret,
    )(ns, w3, p3)
    return out3.reshape(n_rows, d)


def kernel(weight, probs, probs_div, num_samples):
    del probs_div
    return _sparsify(weight, probs, num_samples)


# speedup vs baseline: 1.0621x; 1.0621x over previous
import functools

import jax
import jax.numpy as jnp
import numpy as np
from jax.experimental import pallas as pl
from jax.experimental.pallas import tpu as pltpu

_ROT = ((13, 15, 26, 6), (17, 29, 16, 24))
_KEY1 = 0
_KEY2 = 42
_TINY = np.float32(np.finfo(np.float32).tiny)


def _threefry_bits(x1):
    ks0 = jnp.uint32(_KEY1)
    ks1 = jnp.uint32(_KEY2)
    ks2 = jnp.uint32(_KEY1 ^ _KEY2 ^ 0x1BD11BDA)
    ks = (ks0, ks1, ks2)
    x0 = jnp.zeros_like(x1) + ks0
    for i in range(5):
        for r in _ROT[i % 2]:
            x0 = x0 + x1
            x1 = (x1 << r) | (x1 >> (32 - r))
            x1 = x1 ^ x0
        x0 = x0 + ks[(i + 1) % 3]
        x1 = x1 + ks[(i + 2) % 3] + jnp.uint32(i + 1)
    return x0 ^ x1


def _gumbel(bits):
    fb = (bits >> 9) | jnp.uint32(0x3F800000)
    u = jax.lax.bitcast_convert_type(fb, jnp.float32) - jnp.float32(1.0)
    return -jnp.log(-jnp.log(u))


def _kern(ns_ref, w_ref, p_ref, o_ref, *, rows, d_sub, d_lane, nd_mod, chunk, tail):
    i = pl.program_id(0)
    d = d_sub * d_lane
    d_io = (jax.lax.broadcasted_iota(jnp.uint32, (d_sub, d_lane), 0)
            * jnp.uint32(d_lane)
            + jax.lax.broadcasted_iota(jnp.uint32, (d_sub, d_lane), 1))
    for r in range(rows):
        n = i * rows + r
        ns = ns_ref[n]
        w = w_ref[r]
        logp = jnp.log(p_ref[r] + jnp.float32(1e-30))
        row_ct = d_io + jnp.uint32(d) * jnp.uint32(n) + jnp.uint32(_KEY2)

        def sample(s, rmax, *, masked):
            x1 = row_ct + s.astype(jnp.uint32) * jnp.uint32(nd_mod)
            v = _gumbel(_threefry_bits(x1)) + logp
            m = jnp.max(v, axis=(0, 1), keepdims=True)
            vm = v - m
            if masked:
                vm = jnp.where(s < ns, vm, jnp.float32(-jnp.inf))
            return jnp.maximum(rmax, vm)

        def main_body(j, rmax):
            for c in range(chunk):
                rmax = sample(j * chunk + c, rmax, masked=False)
            return rmax

        n_main = ns // chunk

        def tail_body(j, rmax):
            for c in range(tail):
                rmax = sample(n_main * chunk + j * tail + c, rmax,
                              masked=True)
            return rmax

        rmax = jax.lax.fori_loop(
            0, n_main, main_body,
            jnp.full((d_sub, d_lane), -jnp.inf, jnp.float32))
        rmax = jax.lax.fori_loop(
            0, pl.cdiv(ns - n_main * chunk, tail), tail_body, rmax)
        keep_f = (rmax == 0.0).astype(jnp.float32)
        nnz_c = jnp.sum(keep_f)
        nnz_w = jnp.sum((w != 0).astype(jnp.float32))
        gamma = jnp.where(
            ns >= 1,
            jnp.where(nnz_c < nnz_w, keep_f, jnp.ones_like(keep_f)),
            jnp.zeros_like(keep_f))
        o_ref[r] = gamma * w


def _sparsify(weight, probs, num_samples, rows=8, chunk=32, tail=8, interpret=False):
    n_rows, d = weight.shape
    d_sub = 8
    d_lane = d // d_sub
    w3 = weight.reshape(n_rows, d_sub, d_lane)
    p3 = probs.reshape(n_rows, d_sub, d_lane)
    ns = num_samples.astype(jnp.int32)
    out3 = pl.pallas_call(
        functools.partial(_kern, rows=rows, d_sub=d_sub, d_lane=d_lane,
                          nd_mod=(n_rows * d) % (2 ** 32), chunk=chunk, tail=tail),
        grid_spec=pltpu.PrefetchScalarGridSpec(
            num_scalar_prefetch=1,
            grid=(n_rows // rows,),
            in_specs=[
                pl.BlockSpec((rows, d_sub, d_lane), lambda i, s: (i, 0, 0)),
                pl.BlockSpec((rows, d_sub, d_lane), lambda i, s: (i, 0, 0)),
            ],
            out_specs=pl.BlockSpec((rows, d_sub, d_lane),
                                   lambda i, s: (i, 0, 0)),
        ),
        out_shape=jax.ShapeDtypeStruct((n_rows, d_sub, d_lane), jnp.float32),
        compiler_params=pltpu.CompilerParams(
            dimension_semantics=("parallel",)),
        interpret=interpret,
    )(ns, w3, p3)
    return out3.reshape(n_rows, d)


def kernel(weight, probs, probs_div, num_samples):
    del probs_div
    return _sparsify(weight, probs, num_samples)


# speedup vs baseline: 1.0632x; 1.0010x over previous
import functools

import jax
import jax.numpy as jnp
import numpy as np
from jax.experimental import pallas as pl
from jax.experimental.pallas import tpu as pltpu

_ROT = ((13, 15, 26, 6), (17, 29, 16, 24))
_KEY1 = 0
_KEY2 = 42
_TINY = np.float32(np.finfo(np.float32).tiny)


def _threefry_bits(x1):
    ks0 = jnp.uint32(_KEY1)
    ks1 = jnp.uint32(_KEY2)
    ks2 = jnp.uint32(_KEY1 ^ _KEY2 ^ 0x1BD11BDA)
    ks = (ks0, ks1, ks2)
    x0 = jnp.zeros_like(x1) + ks0
    for i in range(5):
        for r in _ROT[i % 2]:
            x0 = x0 + x1
            x1 = (x1 << r) | (x1 >> (32 - r))
            x1 = x1 ^ x0
        x0 = x0 + ks[(i + 1) % 3]
        x1 = x1 + ks[(i + 2) % 3] + jnp.uint32(i + 1)
    return x0 ^ x1


def _gumbel(bits):
    fb = (bits >> 9) | jnp.uint32(0x3F800000)
    u = jax.lax.bitcast_convert_type(fb, jnp.float32) - jnp.float32(1.0)
    return -jnp.log(-jnp.log(u))


def _kern(ns_ref, w_ref, p_ref, o_ref, *, rows, d_sub, d_lane, nd_mod, chunk, tail):
    i = pl.program_id(0)
    d = d_sub * d_lane
    d_io = (jax.lax.broadcasted_iota(jnp.uint32, (d_sub, d_lane), 0)
            * jnp.uint32(d_lane)
            + jax.lax.broadcasted_iota(jnp.uint32, (d_sub, d_lane), 1))
    for r in range(rows):
        n = i * rows + r
        ns = ns_ref[n]
        w = w_ref[r]
        logp = jnp.log(p_ref[r] + jnp.float32(1e-30))
        row_ct = d_io + jnp.uint32(d) * jnp.uint32(n) + jnp.uint32(_KEY2)

        def sample(s, rmax, *, masked):
            x1 = row_ct + s.astype(jnp.uint32) * jnp.uint32(nd_mod)
            v = _gumbel(_threefry_bits(x1)) + logp
            m = jnp.max(v, axis=(0, 1), keepdims=True)
            vm = v - m
            if masked:
                vm = jnp.where(s < ns, vm, jnp.float32(-jnp.inf))
            return jnp.maximum(rmax, vm)

        def main_body(j, rmax):
            for c in range(chunk):
                rmax = sample(j * chunk + c, rmax, masked=False)
            return rmax

        n_main = ns // chunk

        def tail_body(j, rmax):
            for c in range(tail):
                rmax = sample(n_main * chunk + j * tail + c, rmax,
                              masked=True)
            return rmax

        rmax = jax.lax.fori_loop(
            0, n_main, main_body,
            jnp.full((d_sub, d_lane), -jnp.inf, jnp.float32))
        rmax = jax.lax.fori_loop(
            0, pl.cdiv(ns - n_main * chunk, tail), tail_body, rmax)
        keep_f = (rmax == 0.0).astype(jnp.float32)
        nnz_c = jnp.sum(keep_f)
        nnz_w = jnp.sum((w != 0).astype(jnp.float32))
        gamma = jnp.where(
            ns >= 1,
            jnp.where(nnz_c < nnz_w, keep_f, jnp.ones_like(keep_f)),
            jnp.zeros_like(keep_f))
        o_ref[r] = gamma * w


def _sparsify(weight, probs, num_samples, rows=4, chunk=32, tail=8, interpret=False):
    n_rows, d = weight.shape
    d_sub = 8
    d_lane = d // d_sub
    w3 = weight.reshape(n_rows, d_sub, d_lane)
    p3 = probs.reshape(n_rows, d_sub, d_lane)
    ns = num_samples.astype(jnp.int32)
    out3 = pl.pallas_call(
        functools.partial(_kern, rows=rows, d_sub=d_sub, d_lane=d_lane,
                          nd_mod=(n_rows * d) % (2 ** 32), chunk=chunk, tail=tail),
        grid_spec=pltpu.PrefetchScalarGridSpec(
            num_scalar_prefetch=1,
            grid=(n_rows // rows,),
            in_specs=[
                pl.BlockSpec((rows, d_sub, d_lane), lambda i, s: (i, 0, 0)),
                pl.BlockSpec((rows, d_sub, d_lane), lambda i, s: (i, 0, 0)),
            ],
            out_specs=pl.BlockSpec((rows, d_sub, d_lane),
                                   lambda i, s: (i, 0, 0)),
        ),
        out_shape=jax.ShapeDtypeStruct((n_rows, d_sub, d_lane), jnp.float32),
        compiler_params=pltpu.CompilerParams(
            dimension_semantics=("parallel",)),
        interpret=interpret,
    )(ns, w3, p3)
    return out3.reshape(n_rows, d)


def kernel(weight, probs, probs_div, num_samples):
    del probs_div
    return _sparsify(weight, probs, num_samples)
